# baseline (speedup 1.0000x reference)
import functools

import jax
import jax.numpy as jnp
from jax.experimental import pallas as pl
from jax.experimental.pallas import tpu as pltpu

INPUT_DIM = 256
EMB_DIM = 128
HIDDEN_DIM = 512
BATCH = 128
SEQ = 200
H4 = 4 * HIDDEN_DIM
EMB_CHUNK = 10


def _lstm_kernel(urls_ref, emb_ref, wihf_ref, whhf_ref, bf_ref,
                 wihb_ref, whhb_ref, bb_ref,
                 fcw_ref, fcb_ref, fc1w_ref, fc1b_ref,
                 fc2w_ref, fc2b_ref, fc3w_ref, fc3b_ref,
                 out_ref, aux_ref,
                 x_scr, hf_scr, cf_scr, hb_scr, cb_scr):
    f32 = jnp.float32
    bf16 = jnp.bfloat16

    emb = emb_ref[...].astype(bf16)
    rows = EMB_CHUNK * BATCH
    iota = jax.lax.broadcasted_iota(jnp.int32, (rows, INPUT_DIM), 1)

    def emb_body(k, _):
        ids = urls_ref[pl.ds(k * rows, rows), :]
        x_scr[pl.ds(k * rows, rows), :] = jnp.dot(
            (ids == iota).astype(bf16), emb,
            preferred_element_type=f32).astype(bf16)
        return 0

    jax.lax.fori_loop(0, SEQ * BATCH // rows, emb_body, 0)

    hf_scr[...] = jnp.zeros((BATCH, HIDDEN_DIM), bf16)
    hb_scr[...] = jnp.zeros((BATCH, HIDDEN_DIM), bf16)
    cf_scr[...] = jnp.zeros((BATCH, HIDDEN_DIM), f32)
    cb_scr[...] = jnp.zeros((BATCH, HIDDEN_DIM), f32)

    whhf = whhf_ref[...]
    whhb = whhb_ref[...]
    wihf = wihf_ref[...]
    wihb = wihb_ref[...]
    b_f = bf_ref[...]
    b_b = bb_ref[...]

    def step_dir(x, h, c, wih, whh, b):
        gates = (jnp.dot(x, wih, preferred_element_type=f32)
                 + jnp.dot(h, whh, preferred_element_type=f32) + b)
        i = jax.nn.sigmoid(gates[:, 0:HIDDEN_DIM])
        f = jax.nn.sigmoid(gates[:, HIDDEN_DIM:2 * HIDDEN_DIM])
        g = jnp.tanh(gates[:, 2 * HIDDEN_DIM:3 * HIDDEN_DIM])
        o = jax.nn.sigmoid(gates[:, 3 * HIDDEN_DIM:])
        c = f * c + i * g
        h = (o * jnp.tanh(c)).astype(bf16)
        return h, c

    def body(t, _):
        xf = x_scr[pl.ds(t * BATCH, BATCH), :]
        xb = x_scr[pl.ds((SEQ - 1 - t) * BATCH, BATCH), :]
        hf, cf = step_dir(xf, hf_scr[...], cf_scr[...], wihf, whhf, b_f)
        hb, cb = step_dir(xb, hb_scr[...], cb_scr[...], wihb, whhb, b_b)
        hf_scr[...] = hf
        cf_scr[...] = cf
        hb_scr[...] = hb
        cb_scr[...] = cb
        return 0

    jax.lax.fori_loop(0, SEQ, body, 0)

    hidden = jnp.concatenate([hf_scr[...], hb_scr[...]], axis=1).astype(f32)
    aux_ref[...] = jnp.dot(hidden, fcw_ref[...], preferred_element_type=f32) + fcb_ref[...]
    o1 = jnp.dot(hidden, fc1w_ref[...], preferred_element_type=f32) + fc1b_ref[...]
    o2 = jnp.dot(o1, fc2w_ref[...], preferred_element_type=f32) + fc2b_ref[...]
    out_ref[...] = jnp.dot(o2, fc3w_ref[...], preferred_element_type=f32) + fc3b_ref[...]


@functools.partial(jax.jit, static_argnames=("interpret",))
def _run(urls, emb_table, W_ih_f, W_hh_f, b_f, W_ih_b, W_hh_b, b_b,
         fc_w, fc_b, fc1_w, fc1_b, fc2_w, fc2_b, fc3_w, fc3_b,
         interpret=False):
    urls_flat = urls.T.reshape(SEQ * BATCH, 1).astype(jnp.int32)
    f32 = jnp.float32
    bf16 = jnp.bfloat16
    args = (
        urls_flat,
        emb_table,
        W_ih_f.T.astype(bf16), W_hh_f.T.astype(bf16), b_f.reshape(1, H4),
        W_ih_b.T.astype(bf16), W_hh_b.T.astype(bf16), b_b.reshape(1, H4),
        fc_w.T, fc_b.reshape(1, 1),
        fc1_w.T, fc1_b.reshape(1, H4),
        fc2_w.T, fc2_b.reshape(1, 2 * HIDDEN_DIM),
        fc3_w.T, fc3_b.reshape(1, 2),
    )
    out, aux = pl.pallas_call(
        _lstm_kernel,
        out_shape=(
            jax.ShapeDtypeStruct((BATCH, 2), f32),
            jax.ShapeDtypeStruct((BATCH, 1), f32),
        ),
        scratch_shapes=[
            pltpu.VMEM((SEQ * BATCH, EMB_DIM), bf16),
            pltpu.VMEM((BATCH, HIDDEN_DIM), bf16),
            pltpu.VMEM((BATCH, HIDDEN_DIM), f32),
            pltpu.VMEM((BATCH, HIDDEN_DIM), bf16),
            pltpu.VMEM((BATCH, HIDDEN_DIM), f32),
        ],
        interpret=interpret,
    )(*args)
    return out, aux[:, 0]


def kernel(urls, emb_table, W_ih_f, W_hh_f, b_f, W_ih_b, W_hh_b, b_b,
           fc_w, fc_b, fc1_w, fc1_b, fc2_w, fc2_b, fc3_w, fc3_b):
    return _run(urls, emb_table, W_ih_f, W_hh_f, b_f, W_ih_b, W_hh_b, b_b,
                fc_w, fc_b, fc1_w, fc1_b, fc2_w, fc2_b, fc3_w, fc3_b)
